# baseline (speedup 1.0000x reference)
import functools

import jax
import jax.numpy as jnp
from jax import lax
from jax.experimental import pallas as pl
from jax.experimental.pallas import tpu as pltpu
from jax.experimental.pallas import tpu_sc as plsc

N = 10000
E = 320000
D = 128
H = 64

NC = 2
NS = 16
NW = NC * NS
EPW = E // NW
CH = 125
NCH = EPW // CH
NP = 10240
RPS = NP // NS
ZC = 128

RB = 400
NRB = N // RB

_MESH = plsc.VectorSubcoreMesh(core_axis_name="c", subcore_axis_name="s")


IB = 8
NB = NCH // IB


def _seg_body(F, z_hbm, src2_hbm, dst2_hbm, out_hbm, sidx, didx, rows, acc):
    pl.run_scoped(
        functools.partial(_seg_inner, F, z_hbm, src2_hbm, dst2_hbm,
                          out_hbm, sidx, didx, rows, acc),
        *([pltpu.SemaphoreType.DMA] * 8))


def _seg_inner(F, z_hbm, src2_hbm, dst2_hbm, out_hbm, sidx, didx, rows,
               acc, gs0, gs1, ss0, ss1, is0, is1, id0, id1):
    zsrc0 = rows.at[0]
    c = lax.axis_index("c")
    s = lax.axis_index("s")
    wid = s * NC + c
    gsem = (gs0, gs1)
    ssem = (ss0, ss1)
    isem_s = (is0, is1)
    isem_d = (id0, id1)

    zv = jnp.zeros((16,), jnp.float32)
    kpr = F // 16

    def zb(i, carry):
        for k in range(kpr):
            zsrc0[i, pl.ds(k * 16, 16)] = zv
        return carry

    lax.fori_loop(0, CH, zb, 0)

    zsrc = zsrc0
    nz = RPS // CH
    for i in range(nz):
        pltpu.async_copy(zsrc, acc.at[pl.ds(s * RPS + i * CH, CH)],
                         ssem[i % 2])
    ztail_src = zsrc.at[pl.ds(0, RPS % CH)]
    ztail_dst = acc.at[pl.ds(s * RPS + nz * CH, RPS % CH)]
    pltpu.async_copy(ztail_src, ztail_dst, ssem[nz % 2])

    rv = (rows.at[0], rows.at[1])
    sv = rv

    def i_start(m, slot, sem_pair):
        pltpu.async_copy(src2_hbm.at[pl.ds((wid * NB + m) * IB, IB)],
                         sidx.at[slot], sem_pair[0])
        pltpu.async_copy(dst2_hbm.at[pl.ds((wid * NB + m) * IB, IB)],
                         didx.at[slot], sem_pair[1])

    def i_wait(m, slot, sem_pair):
        pltpu.make_async_copy(src2_hbm.at[pl.ds((wid * NB + m) * IB, IB)],
                              sidx.at[slot], sem_pair[0]).wait()
        pltpu.make_async_copy(dst2_hbm.at[pl.ds((wid * NB + m) * IB, IB)],
                              didx.at[slot], sem_pair[1]).wait()

    def g_start(srow, rb):
        pltpu.async_copy(z_hbm.at[srow], rv[rb], gsem[rb])

    def g_wait(srow, rb):
        pltpu.make_async_copy(z_hbm.at[srow], rv[rb], gsem[rb]).wait()

    def s_start(drow, rb):
        pltpu.async_copy(sv[rb], acc.at[drow], ssem[rb], add=True)

    def s_wait(drow, rb):
        pltpu.make_async_copy(sv[rb], acc.at[drow], ssem[rb]).wait()

    pltpu.sync_copy(src2_hbm.at[pl.ds(wid * NB * IB, IB)], sidx.at[0])
    pltpu.sync_copy(dst2_hbm.at[pl.ds(wid * NB * IB, IB)], didx.at[0])
    i_start(1, 1, (isem_s[1], isem_d[1]))
    for i in range(nz):
        pltpu.make_async_copy(zsrc, acc.at[pl.ds(s * RPS + i * CH, CH)],
                              ssem[i % 2]).wait()
    pltpu.make_async_copy(ztail_src, ztail_dst, ssem[nz % 2]).wait()
    plsc.subcore_barrier()
    g_start(sidx.at[0].at[0], 0)

    def outer(bi2, carry):
        for p in range(2):
            m = bi2 * 2 + p
            for b in range(IB):
                j = m * IB + b
                rb = b % 2
                g_wait(sidx.at[p].at[b], rb)
                s_start(didx.at[p].at[b], rb)

                @pl.when(j >= 1)
                def _():
                    s_wait(didx.at[p].at[b], 1 - rb)

                if b == 0:
                    @pl.when((m >= 1) & (m + 1 < NB))
                    def _():
                        i_start(m + 1, 1 - p, (isem_s[1 - p], isem_d[1 - p]))

                if b == IB - 1:
                    @pl.when(m + 1 < NB)
                    def _():
                        i_wait(m + 1, 1 - p, (isem_s[1 - p], isem_d[1 - p]))
                        g_start(sidx.at[1 - p].at[0], 1 - rb)
                else:
                    @pl.when(j + 1 < NCH)
                    def _():
                        g_start(sidx.at[p].at[b + 1], 1 - rb)
        return carry

    lax.fori_loop(0, NB // 2, outer, 0)
    s_wait(didx.at[1].at[IB - 1], (NCH - 1) % 2)
    plsc.subcore_barrier()
    pltpu.sync_copy(acc.at[pl.ds(s * RPS, RPS)],
                    out_hbm.at[c].at[pl.ds(s * RPS, RPS)])


def _make_segsum(F):
    return functools.partial(
        pl.kernel,
        out_type=jax.ShapeDtypeStruct((NC, NP, F), jnp.float32),
        mesh=_MESH,
        scratch_types=[
            pltpu.VMEM((2, IB, CH), jnp.int32),
            pltpu.VMEM((2, IB, CH), jnp.int32),
            pltpu.VMEM((2, CH, F), jnp.float32),
            pltpu.VMEM_SHARED((NP, F), jnp.float32),
        ],
    )(functools.partial(_seg_body, F))


_seg128 = _make_segsum(D)


@functools.partial(
    pl.kernel,
    out_type=jax.ShapeDtypeStruct((NC, NP), jnp.float32),
    mesh=_MESH,
    scratch_types=[
        pltpu.VMEM((NCH, CH), jnp.int32),
        pltpu.VMEM((128,), jnp.float32),
        pltpu.VMEM((RPS,), jnp.float32),
        pltpu.VMEM_SHARED((NP,), jnp.float32),
    ],
)
def _deg(dst2_hbm, out_hbm, didx, ones, zbuf, acc):
    c = lax.axis_index("c")
    s = lax.axis_index("s")
    wid = s * NC + c

    ov = jnp.ones((16,), jnp.float32)
    zv = jnp.zeros((16,), jnp.float32)

    def fo(i, carry):
        ones[pl.ds(i * 16, 16)] = ov
        return carry

    lax.fori_loop(0, 128 // 16, fo, 0)

    def fz(i, carry):
        zbuf[pl.ds(i * 16, 16)] = zv
        return carry

    lax.fori_loop(0, RPS // 16, fz, 0)
    pltpu.sync_copy(zbuf, acc.at[pl.ds(s * RPS, RPS)])
    plsc.subcore_barrier()

    pltpu.sync_copy(dst2_hbm.at[pl.ds(wid * NCH, NCH)], didx)

    def body(j, carry):
        pltpu.sync_copy(ones.at[pl.ds(0, CH)], acc.at[didx.at[j]], add=True)
        return carry

    lax.fori_loop(0, NCH, body, 0)
    plsc.subcore_barrier()
    pltpu.sync_copy(acc.at[pl.ds(s * RPS, RPS)],
                    out_hbm.at[c].at[pl.ds(s * RPS, RPS)])


def _tc_dis_u_body(degp_ref, x_ref, dis_ref, u_ref):
    deg = degp_ref[0] + degp_ref[1]
    dis = jnp.where(deg > 0, lax.rsqrt(jnp.maximum(deg, 1e-12)), 0.0)
    dis_ref[...] = dis
    u_ref[...] = x_ref[...] * dis


def _tc_dis_u(degp3, x):
    return pl.pallas_call(
        _tc_dis_u_body,
        grid=(NRB,),
        in_specs=[
            pl.BlockSpec((NC, RB, 1), lambda b: (0, b, 0)),
            pl.BlockSpec((RB, D), lambda b: (b, 0)),
        ],
        out_specs=[
            pl.BlockSpec((RB, 1), lambda b: (b, 0)),
            pl.BlockSpec((RB, D), lambda b: (b, 0)),
        ],
        out_shape=[
            jax.ShapeDtypeStruct((N, 1), jnp.float32),
            jax.ShapeDtypeStruct((N, D), jnp.float32),
        ],
    )(degp3, x)


def _tc_mid_body(pad, gp_ref, dis_ref, tx1_ref, u2_ref):
    g = gp_ref[0] + gp_ref[1]
    dis = dis_ref[...]
    t = -dis * g
    tx1_ref[...] = t
    u2 = dis * t
    if pad:
        u2_ref[...] = jnp.concatenate([u2, jnp.zeros_like(u2)], axis=1)
    else:
        u2_ref[...] = u2


def _tc_mid(gp, dis, F, pad=False):
    FO = 2 * F if pad else F
    return pl.pallas_call(
        functools.partial(_tc_mid_body, pad),
        grid=(NRB,),
        in_specs=[
            pl.BlockSpec((NC, RB, F), lambda b: (0, b, 0)),
            pl.BlockSpec((RB, 1), lambda b: (b, 0)),
        ],
        out_specs=[
            pl.BlockSpec((RB, F), lambda b: (b, 0)),
            pl.BlockSpec((RB, FO), lambda b: (b, 0)),
        ],
        out_shape=[
            jax.ShapeDtypeStruct((N, F), jnp.float32),
            jax.ShapeDtypeStruct((N, FO), jnp.float32),
        ],
    )(gp, dis)


def _tc_fin1_body(z_ref, tx1_ref, g2p_ref, dis_ref, w_ref, b_ref,
                  h_ref, u_ref):
    z = z_ref[...]
    dis = dis_ref[...]
    tx2 = -2.0 * dis * (g2p_ref[0] + g2p_ref[1]) - z
    o = (jnp.dot(z, w_ref[0], preferred_element_type=jnp.float32)
         + jnp.dot(tx1_ref[...], w_ref[1], preferred_element_type=jnp.float32)
         + jnp.dot(tx2, w_ref[2], preferred_element_type=jnp.float32)
         + b_ref[...])
    hh = jnp.maximum(o, 0.0)
    zpad = jnp.zeros_like(hh)
    h_ref[...] = jnp.concatenate([hh, zpad], axis=1)
    u_ref[...] = jnp.concatenate([dis * hh, zpad], axis=1)


def _tc_fin1(x, tx1, g2p, dis, W1, b1):
    return pl.pallas_call(
        _tc_fin1_body,
        grid=(NRB,),
        in_specs=[
            pl.BlockSpec((RB, D), lambda b: (b, 0)),
            pl.BlockSpec((RB, D), lambda b: (b, 0)),
            pl.BlockSpec((NC, RB, D), lambda b: (0, b, 0)),
            pl.BlockSpec((RB, 1), lambda b: (b, 0)),
            pl.BlockSpec((3, D, H), lambda b: (0, 0, 0)),
            pl.BlockSpec((1, H), lambda b: (0, 0)),
        ],
        out_specs=[
            pl.BlockSpec((RB, D), lambda b: (b, 0)),
            pl.BlockSpec((RB, D), lambda b: (b, 0)),
        ],
        out_shape=[
            jax.ShapeDtypeStruct((N, D), jnp.float32),
            jax.ShapeDtypeStruct((N, D), jnp.float32),
        ],
    )(x, tx1, g2p, dis, W1, b1)


def _tc_fin2_body(h_ref, ty1_ref, g2p_ref, dis_ref, w_ref, b_ref, x_ref,
                  xhat_ref, loss_ref):
    h = h_ref[...]
    dis = dis_ref[...]
    ty2 = -2.0 * dis * (g2p_ref[0] + g2p_ref[1]) - h
    o = (jnp.dot(h, w_ref[0], preferred_element_type=jnp.float32)
         + jnp.dot(ty1_ref[...], w_ref[1], preferred_element_type=jnp.float32)
         + jnp.dot(ty2, w_ref[2], preferred_element_type=jnp.float32)
         + b_ref[...])
    xhat_ref[...] = o
    d = o - x_ref[...]
    b_idx = pl.program_id(0)

    @pl.when(b_idx == 0)
    def _():
        loss_ref[...] = jnp.zeros((1, 1), jnp.float32)

    loss_ref[...] += jnp.sum(d * d).reshape(1, 1)

    @pl.when(b_idx == NRB - 1)
    def _():
        loss_ref[...] = loss_ref[...] * (1.0 / (N * D))


def _tc_fin2(h, ty1, g2p, dis, W2, b2, x):
    return pl.pallas_call(
        _tc_fin2_body,
        grid=(NRB,),
        in_specs=[
            pl.BlockSpec((RB, D), lambda b: (b, 0)),
            pl.BlockSpec((RB, D), lambda b: (b, 0)),
            pl.BlockSpec((NC, RB, D), lambda b: (0, b, 0)),
            pl.BlockSpec((RB, 1), lambda b: (b, 0)),
            pl.BlockSpec((3, D, D), lambda b: (0, 0, 0)),
            pl.BlockSpec((1, D), lambda b: (0, 0)),
            pl.BlockSpec((RB, D), lambda b: (b, 0)),
        ],
        out_specs=[
            pl.BlockSpec((RB, D), lambda b: (b, 0)),
            pl.BlockSpec((1, 1), lambda b: (0, 0)),
        ],
        out_shape=[
            jax.ShapeDtypeStruct((N, D), jnp.float32),
            jax.ShapeDtypeStruct((1, 1), jnp.float32),
        ],
    )(h, ty1, g2p, dis, W2, b2, x)


def kernel(x, edge_index, W1, b1, W2, b2):
    src2 = edge_index[0].reshape(NW * NCH, CH)
    dst2 = edge_index[1].reshape(NW * NCH, CH)

    degp = _deg(dst2)
    degp3 = degp[:, :N].reshape(NC, N, 1)
    dis, u = _tc_dis_u(degp3, x)

    g1p = _seg128(u, src2, dst2)
    tx1, u2 = _tc_mid(g1p, dis, D)
    g2p = _seg128(u2, src2, dst2)
    h, uh = _tc_fin1(x, tx1, g2p, dis, W1, b1.reshape(1, H))

    W2p = jnp.concatenate([W2, jnp.zeros((3, D - H, D), jnp.float32)], axis=1)

    q1p = _seg128(uh, src2, dst2)
    ty1, v2 = _tc_mid(q1p, dis, D)
    q2p = _seg128(v2, src2, dst2)
    xhat, lossb = _tc_fin2(h, ty1, q2p, dis, W2p, b2.reshape(1, D), x)

    return (xhat, lossb[0, 0])


# speedup vs baseline: 1.0769x; 1.0769x over previous
import functools

import jax
import jax.numpy as jnp
from jax import lax
from jax.experimental import pallas as pl
from jax.experimental.pallas import tpu as pltpu
from jax.experimental.pallas import tpu_sc as plsc

N = 10000
E = 320000
D = 128
H = 64

NC = 2
NS = 16
NW = NC * NS
EPW = E // NW
CH = 125
NCH = EPW // CH
NP = 10240
RPS = NP // NS
ZC = 128

RB = 2000
NRB = N // RB

_MESH = plsc.VectorSubcoreMesh(core_axis_name="c", subcore_axis_name="s")


IB = 8
NB = NCH // IB


def _seg_body(F, z_hbm, src2_hbm, dst2_hbm, out_hbm, sidx, didx, rows, acc):
    pl.run_scoped(
        functools.partial(_seg_inner, F, z_hbm, src2_hbm, dst2_hbm,
                          out_hbm, sidx, didx, rows, acc),
        *([pltpu.SemaphoreType.DMA] * 8))


def _seg_inner(F, z_hbm, src2_hbm, dst2_hbm, out_hbm, sidx, didx, rows,
               acc, gs0, gs1, ss0, ss1, is0, is1, id0, id1):
    zsrc0 = rows.at[0]
    c = lax.axis_index("c")
    s = lax.axis_index("s")
    wid = s * NC + c
    gsem = (gs0, gs1)
    ssem = (ss0, ss1)
    isem_s = (is0, is1)
    isem_d = (id0, id1)

    zv = jnp.zeros((16,), jnp.float32)
    kpr = F // 16

    def zb(i, carry):
        for k in range(kpr):
            zsrc0[i, pl.ds(k * 16, 16)] = zv
        return carry

    lax.fori_loop(0, CH, zb, 0)

    zsrc = zsrc0
    nz = RPS // CH
    for i in range(nz):
        pltpu.async_copy(zsrc, acc.at[pl.ds(s * RPS + i * CH, CH)],
                         ssem[i % 2])
    ztail_src = zsrc.at[pl.ds(0, RPS % CH)]
    ztail_dst = acc.at[pl.ds(s * RPS + nz * CH, RPS % CH)]
    pltpu.async_copy(ztail_src, ztail_dst, ssem[nz % 2])

    rv = (rows.at[0], rows.at[1])
    sv = rv

    def i_start(m, slot, sem_pair):
        pltpu.async_copy(src2_hbm.at[pl.ds((wid * NB + m) * IB, IB)],
                         sidx.at[slot], sem_pair[0])
        pltpu.async_copy(dst2_hbm.at[pl.ds((wid * NB + m) * IB, IB)],
                         didx.at[slot], sem_pair[1])

    def i_wait(m, slot, sem_pair):
        pltpu.make_async_copy(src2_hbm.at[pl.ds((wid * NB + m) * IB, IB)],
                              sidx.at[slot], sem_pair[0]).wait()
        pltpu.make_async_copy(dst2_hbm.at[pl.ds((wid * NB + m) * IB, IB)],
                              didx.at[slot], sem_pair[1]).wait()

    def g_start(srow, rb):
        pltpu.async_copy(z_hbm.at[srow], rv[rb], gsem[rb])

    def g_wait(srow, rb):
        pltpu.make_async_copy(z_hbm.at[srow], rv[rb], gsem[rb]).wait()

    def s_start(drow, rb):
        pltpu.async_copy(sv[rb], acc.at[drow], ssem[rb], add=True)

    def s_wait(drow, rb):
        pltpu.make_async_copy(sv[rb], acc.at[drow], ssem[rb]).wait()

    pltpu.sync_copy(src2_hbm.at[pl.ds(wid * NB * IB, IB)], sidx.at[0])
    pltpu.sync_copy(dst2_hbm.at[pl.ds(wid * NB * IB, IB)], didx.at[0])
    i_start(1, 1, (isem_s[1], isem_d[1]))
    for i in range(nz):
        pltpu.make_async_copy(zsrc, acc.at[pl.ds(s * RPS + i * CH, CH)],
                              ssem[i % 2]).wait()
    pltpu.make_async_copy(ztail_src, ztail_dst, ssem[nz % 2]).wait()
    plsc.subcore_barrier()
    g_start(sidx.at[0].at[0], 0)

    def outer(bi2, carry):
        for p in range(2):
            m = bi2 * 2 + p
            for b in range(IB):
                j = m * IB + b
                rb = b % 2
                g_wait(sidx.at[p].at[b], rb)
                s_start(didx.at[p].at[b], rb)

                @pl.when(j >= 1)
                def _():
                    s_wait(didx.at[p].at[b], 1 - rb)

                if b == 0:
                    @pl.when((m >= 1) & (m + 1 < NB))
                    def _():
                        i_start(m + 1, 1 - p, (isem_s[1 - p], isem_d[1 - p]))

                if b == IB - 1:
                    @pl.when(m + 1 < NB)
                    def _():
                        i_wait(m + 1, 1 - p, (isem_s[1 - p], isem_d[1 - p]))
                        g_start(sidx.at[1 - p].at[0], 1 - rb)
                else:
                    @pl.when(j + 1 < NCH)
                    def _():
                        g_start(sidx.at[p].at[b + 1], 1 - rb)
        return carry

    lax.fori_loop(0, NB // 2, outer, 0)
    s_wait(didx.at[1].at[IB - 1], (NCH - 1) % 2)
    plsc.subcore_barrier()
    pltpu.sync_copy(acc.at[pl.ds(s * RPS, RPS)],
                    out_hbm.at[c].at[pl.ds(s * RPS, RPS)])


def _make_segsum(F):
    return functools.partial(
        pl.kernel,
        out_type=jax.ShapeDtypeStruct((NC, NP, F), jnp.float32),
        mesh=_MESH,
        scratch_types=[
            pltpu.VMEM((2, IB, CH), jnp.int32),
            pltpu.VMEM((2, IB, CH), jnp.int32),
            pltpu.VMEM((2, CH, F), jnp.float32),
            pltpu.VMEM_SHARED((NP, F), jnp.float32),
        ],
    )(functools.partial(_seg_body, F))


_seg128 = _make_segsum(D)


@functools.partial(
    pl.kernel,
    out_type=jax.ShapeDtypeStruct((NC, NP), jnp.float32),
    mesh=_MESH,
    scratch_types=[
        pltpu.VMEM((NCH, CH), jnp.int32),
        pltpu.VMEM((128,), jnp.float32),
        pltpu.VMEM((RPS,), jnp.float32),
        pltpu.VMEM_SHARED((NP,), jnp.float32),
    ],
)
def _deg(dst2_hbm, out_hbm, didx, ones, zbuf, acc):
    c = lax.axis_index("c")
    s = lax.axis_index("s")
    wid = s * NC + c

    ov = jnp.ones((16,), jnp.float32)
    zv = jnp.zeros((16,), jnp.float32)

    def fo(i, carry):
        ones[pl.ds(i * 16, 16)] = ov
        return carry

    lax.fori_loop(0, 128 // 16, fo, 0)

    def fz(i, carry):
        zbuf[pl.ds(i * 16, 16)] = zv
        return carry

    lax.fori_loop(0, RPS // 16, fz, 0)
    pltpu.sync_copy(zbuf, acc.at[pl.ds(s * RPS, RPS)])
    plsc.subcore_barrier()

    pltpu.sync_copy(dst2_hbm.at[pl.ds(wid * NCH, NCH)], didx)

    def body(j, carry):
        pltpu.sync_copy(ones.at[pl.ds(0, CH)], acc.at[didx.at[j]], add=True)
        return carry

    lax.fori_loop(0, NCH, body, 0)
    plsc.subcore_barrier()
    pltpu.sync_copy(acc.at[pl.ds(s * RPS, RPS)],
                    out_hbm.at[c].at[pl.ds(s * RPS, RPS)])


def _tc_dis_u_body(degp_ref, x_ref, dis_ref, u_ref):
    deg = degp_ref[0] + degp_ref[1]
    dis = jnp.where(deg > 0, lax.rsqrt(jnp.maximum(deg, 1e-12)), 0.0)
    dis_ref[...] = dis
    u_ref[...] = x_ref[...] * dis


def _tc_dis_u(degp3, x):
    return pl.pallas_call(
        _tc_dis_u_body,
        grid=(NRB,),
        in_specs=[
            pl.BlockSpec((NC, RB, 1), lambda b: (0, b, 0)),
            pl.BlockSpec((RB, D), lambda b: (b, 0)),
        ],
        out_specs=[
            pl.BlockSpec((RB, 1), lambda b: (b, 0)),
            pl.BlockSpec((RB, D), lambda b: (b, 0)),
        ],
        out_shape=[
            jax.ShapeDtypeStruct((N, 1), jnp.float32),
            jax.ShapeDtypeStruct((N, D), jnp.float32),
        ],
    )(degp3, x)


def _tc_mid_body(pad, gp_ref, dis_ref, tx1_ref, u2_ref):
    g = gp_ref[0] + gp_ref[1]
    dis = dis_ref[...]
    t = -dis * g
    tx1_ref[...] = t
    u2 = dis * t
    if pad:
        u2_ref[...] = jnp.concatenate([u2, jnp.zeros_like(u2)], axis=1)
    else:
        u2_ref[...] = u2


def _tc_mid(gp, dis, F, pad=False):
    FO = 2 * F if pad else F
    return pl.pallas_call(
        functools.partial(_tc_mid_body, pad),
        grid=(NRB,),
        in_specs=[
            pl.BlockSpec((NC, RB, F), lambda b: (0, b, 0)),
            pl.BlockSpec((RB, 1), lambda b: (b, 0)),
        ],
        out_specs=[
            pl.BlockSpec((RB, F), lambda b: (b, 0)),
            pl.BlockSpec((RB, FO), lambda b: (b, 0)),
        ],
        out_shape=[
            jax.ShapeDtypeStruct((N, F), jnp.float32),
            jax.ShapeDtypeStruct((N, FO), jnp.float32),
        ],
    )(gp, dis)


def _tc_fin1_body(z_ref, tx1_ref, g2p_ref, dis_ref, w_ref, b_ref,
                  h_ref, u_ref):
    z = z_ref[...]
    dis = dis_ref[...]
    tx2 = -2.0 * dis * (g2p_ref[0] + g2p_ref[1]) - z
    o = (jnp.dot(z, w_ref[0], preferred_element_type=jnp.float32)
         + jnp.dot(tx1_ref[...], w_ref[1], preferred_element_type=jnp.float32)
         + jnp.dot(tx2, w_ref[2], preferred_element_type=jnp.float32)
         + b_ref[...])
    hh = jnp.maximum(o, 0.0)
    zpad = jnp.zeros_like(hh)
    h_ref[...] = jnp.concatenate([hh, zpad], axis=1)
    u_ref[...] = jnp.concatenate([dis * hh, zpad], axis=1)


def _tc_fin1(x, tx1, g2p, dis, W1, b1):
    return pl.pallas_call(
        _tc_fin1_body,
        grid=(NRB,),
        in_specs=[
            pl.BlockSpec((RB, D), lambda b: (b, 0)),
            pl.BlockSpec((RB, D), lambda b: (b, 0)),
            pl.BlockSpec((NC, RB, D), lambda b: (0, b, 0)),
            pl.BlockSpec((RB, 1), lambda b: (b, 0)),
            pl.BlockSpec((3, D, H), lambda b: (0, 0, 0)),
            pl.BlockSpec((1, H), lambda b: (0, 0)),
        ],
        out_specs=[
            pl.BlockSpec((RB, D), lambda b: (b, 0)),
            pl.BlockSpec((RB, D), lambda b: (b, 0)),
        ],
        out_shape=[
            jax.ShapeDtypeStruct((N, D), jnp.float32),
            jax.ShapeDtypeStruct((N, D), jnp.float32),
        ],
    )(x, tx1, g2p, dis, W1, b1)


def _tc_fin2_body(h_ref, ty1_ref, g2p_ref, dis_ref, w_ref, b_ref, x_ref,
                  xhat_ref, loss_ref):
    h = h_ref[...]
    dis = dis_ref[...]
    ty2 = -2.0 * dis * (g2p_ref[0] + g2p_ref[1]) - h
    o = (jnp.dot(h, w_ref[0], preferred_element_type=jnp.float32)
         + jnp.dot(ty1_ref[...], w_ref[1], preferred_element_type=jnp.float32)
         + jnp.dot(ty2, w_ref[2], preferred_element_type=jnp.float32)
         + b_ref[...])
    xhat_ref[...] = o
    d = o - x_ref[...]
    b_idx = pl.program_id(0)

    @pl.when(b_idx == 0)
    def _():
        loss_ref[...] = jnp.zeros((1, 1), jnp.float32)

    loss_ref[...] += jnp.sum(d * d).reshape(1, 1)

    @pl.when(b_idx == NRB - 1)
    def _():
        loss_ref[...] = loss_ref[...] * (1.0 / (N * D))


def _tc_fin2(h, ty1, g2p, dis, W2, b2, x):
    return pl.pallas_call(
        _tc_fin2_body,
        grid=(NRB,),
        in_specs=[
            pl.BlockSpec((RB, D), lambda b: (b, 0)),
            pl.BlockSpec((RB, D), lambda b: (b, 0)),
            pl.BlockSpec((NC, RB, D), lambda b: (0, b, 0)),
            pl.BlockSpec((RB, 1), lambda b: (b, 0)),
            pl.BlockSpec((3, D, D), lambda b: (0, 0, 0)),
            pl.BlockSpec((1, D), lambda b: (0, 0)),
            pl.BlockSpec((RB, D), lambda b: (b, 0)),
        ],
        out_specs=[
            pl.BlockSpec((RB, D), lambda b: (b, 0)),
            pl.BlockSpec((1, 1), lambda b: (0, 0)),
        ],
        out_shape=[
            jax.ShapeDtypeStruct((N, D), jnp.float32),
            jax.ShapeDtypeStruct((1, 1), jnp.float32),
        ],
    )(h, ty1, g2p, dis, W2, b2, x)


def kernel(x, edge_index, W1, b1, W2, b2):
    src2 = edge_index[0].reshape(NW * NCH, CH)
    dst2 = edge_index[1].reshape(NW * NCH, CH)

    degp = _deg(dst2)
    degp3 = degp[:, :N].reshape(NC, N, 1)
    dis, u = _tc_dis_u(degp3, x)

    g1p = _seg128(u, src2, dst2)
    tx1, u2 = _tc_mid(g1p, dis, D)
    g2p = _seg128(u2, src2, dst2)
    h, uh = _tc_fin1(x, tx1, g2p, dis, W1, b1.reshape(1, H))

    W2p = jnp.concatenate([W2, jnp.zeros((3, D - H, D), jnp.float32)], axis=1)

    q1p = _seg128(uh, src2, dst2)
    ty1, v2 = _tc_mid(q1p, dis, D)
    q2p = _seg128(v2, src2, dst2)
    xhat, lossb = _tc_fin2(h, ty1, q2p, dis, W2p, b2.reshape(1, D), x)

    return (xhat, lossb[0, 0])


# speedup vs baseline: 1.0916x; 1.0137x over previous
import functools

import jax
import jax.numpy as jnp
from jax import lax
from jax.experimental import pallas as pl
from jax.experimental.pallas import tpu as pltpu
from jax.experimental.pallas import tpu_sc as plsc

N = 10000
E = 320000
D = 128
H = 64

NC = 2
NS = 16
NW = NC * NS
EPW = E // NW
CH = 125
NCH = EPW // CH
NP = 10240
RPS = NP // NS
ZC = 128

RB = 5000
NRB = N // RB

_MESH = plsc.VectorSubcoreMesh(core_axis_name="c", subcore_axis_name="s")


IB = 8
NB = NCH // IB


def _seg_body(F, z_hbm, src2_hbm, dst2_hbm, out_hbm, sidx, didx, rows, acc):
    pl.run_scoped(
        functools.partial(_seg_inner, F, z_hbm, src2_hbm, dst2_hbm,
                          out_hbm, sidx, didx, rows, acc),
        *([pltpu.SemaphoreType.DMA] * 8))


def _seg_inner(F, z_hbm, src2_hbm, dst2_hbm, out_hbm, sidx, didx, rows,
               acc, gs0, gs1, ss0, ss1, is0, is1, id0, id1):
    zsrc0 = rows.at[0]
    c = lax.axis_index("c")
    s = lax.axis_index("s")
    wid = s * NC + c
    gsem = (gs0, gs1)
    ssem = (ss0, ss1)
    isem_s = (is0, is1)
    isem_d = (id0, id1)

    zv = jnp.zeros((16,), jnp.float32)
    kpr = F // 16

    def zb(i, carry):
        for k in range(kpr):
            zsrc0[i, pl.ds(k * 16, 16)] = zv
        return carry

    lax.fori_loop(0, CH, zb, 0)

    zsrc = zsrc0
    nz = RPS // CH
    for i in range(nz):
        pltpu.async_copy(zsrc, acc.at[pl.ds(s * RPS + i * CH, CH)],
                         ssem[i % 2])
    ztail_src = zsrc.at[pl.ds(0, RPS % CH)]
    ztail_dst = acc.at[pl.ds(s * RPS + nz * CH, RPS % CH)]
    pltpu.async_copy(ztail_src, ztail_dst, ssem[nz % 2])

    rv = (rows.at[0], rows.at[1])
    sv = rv

    def i_start(m, slot, sem_pair):
        pltpu.async_copy(src2_hbm.at[pl.ds((wid * NB + m) * IB, IB)],
                         sidx.at[slot], sem_pair[0])
        pltpu.async_copy(dst2_hbm.at[pl.ds((wid * NB + m) * IB, IB)],
                         didx.at[slot], sem_pair[1])

    def i_wait(m, slot, sem_pair):
        pltpu.make_async_copy(src2_hbm.at[pl.ds((wid * NB + m) * IB, IB)],
                              sidx.at[slot], sem_pair[0]).wait()
        pltpu.make_async_copy(dst2_hbm.at[pl.ds((wid * NB + m) * IB, IB)],
                              didx.at[slot], sem_pair[1]).wait()

    def g_start(srow, rb):
        pltpu.async_copy(z_hbm.at[srow], rv[rb], gsem[rb])

    def g_wait(srow, rb):
        pltpu.make_async_copy(z_hbm.at[srow], rv[rb], gsem[rb]).wait()

    def s_start(drow, rb):
        pltpu.async_copy(sv[rb], acc.at[drow], ssem[rb], add=True)

    def s_wait(drow, rb):
        pltpu.make_async_copy(sv[rb], acc.at[drow], ssem[rb]).wait()

    pltpu.sync_copy(src2_hbm.at[pl.ds(wid * NB * IB, IB)], sidx.at[0])
    pltpu.sync_copy(dst2_hbm.at[pl.ds(wid * NB * IB, IB)], didx.at[0])
    i_start(1, 1, (isem_s[1], isem_d[1]))
    for i in range(nz):
        pltpu.make_async_copy(zsrc, acc.at[pl.ds(s * RPS + i * CH, CH)],
                              ssem[i % 2]).wait()
    pltpu.make_async_copy(ztail_src, ztail_dst, ssem[nz % 2]).wait()
    plsc.subcore_barrier()
    g_start(sidx.at[0].at[0], 0)

    def outer(bi2, carry):
        for p in range(2):
            m = bi2 * 2 + p
            for b in range(IB):
                j = m * IB + b
                rb = b % 2
                g_wait(sidx.at[p].at[b], rb)
                s_start(didx.at[p].at[b], rb)

                @pl.when(j >= 1)
                def _():
                    s_wait(didx.at[p].at[b], 1 - rb)

                if b == 0:
                    @pl.when((m >= 1) & (m + 1 < NB))
                    def _():
                        i_start(m + 1, 1 - p, (isem_s[1 - p], isem_d[1 - p]))

                if b == IB - 1:
                    @pl.when(m + 1 < NB)
                    def _():
                        i_wait(m + 1, 1 - p, (isem_s[1 - p], isem_d[1 - p]))
                        g_start(sidx.at[1 - p].at[0], 1 - rb)
                else:
                    @pl.when(j + 1 < NCH)
                    def _():
                        g_start(sidx.at[p].at[b + 1], 1 - rb)
        return carry

    lax.fori_loop(0, NB // 2, outer, 0)
    s_wait(didx.at[1].at[IB - 1], (NCH - 1) % 2)
    plsc.subcore_barrier()
    pltpu.sync_copy(acc.at[pl.ds(s * RPS, RPS)],
                    out_hbm.at[c].at[pl.ds(s * RPS, RPS)])


def _make_segsum(F):
    return functools.partial(
        pl.kernel,
        out_type=jax.ShapeDtypeStruct((NC, NP, F), jnp.float32),
        mesh=_MESH,
        scratch_types=[
            pltpu.VMEM((2, IB, CH), jnp.int32),
            pltpu.VMEM((2, IB, CH), jnp.int32),
            pltpu.VMEM((2, CH, F), jnp.float32),
            pltpu.VMEM_SHARED((NP, F), jnp.float32),
        ],
    )(functools.partial(_seg_body, F))


_seg128 = _make_segsum(D)


@functools.partial(
    pl.kernel,
    out_type=jax.ShapeDtypeStruct((NC, NP), jnp.float32),
    mesh=_MESH,
    scratch_types=[
        pltpu.VMEM((NCH, CH), jnp.int32),
        pltpu.VMEM((128,), jnp.float32),
        pltpu.VMEM((RPS,), jnp.float32),
        pltpu.VMEM_SHARED((NP,), jnp.float32),
    ],
)
def _deg(dst2_hbm, out_hbm, didx, ones, zbuf, acc):
    c = lax.axis_index("c")
    s = lax.axis_index("s")
    wid = s * NC + c

    ov = jnp.ones((16,), jnp.float32)
    zv = jnp.zeros((16,), jnp.float32)

    def fo(i, carry):
        ones[pl.ds(i * 16, 16)] = ov
        return carry

    lax.fori_loop(0, 128 // 16, fo, 0)

    def fz(i, carry):
        zbuf[pl.ds(i * 16, 16)] = zv
        return carry

    lax.fori_loop(0, RPS // 16, fz, 0)
    pltpu.sync_copy(zbuf, acc.at[pl.ds(s * RPS, RPS)])
    plsc.subcore_barrier()

    pltpu.sync_copy(dst2_hbm.at[pl.ds(wid * NCH, NCH)], didx)

    def body(j, carry):
        pltpu.sync_copy(ones.at[pl.ds(0, CH)], acc.at[didx.at[j]], add=True)
        return carry

    lax.fori_loop(0, NCH, body, 0)
    plsc.subcore_barrier()
    pltpu.sync_copy(acc.at[pl.ds(s * RPS, RPS)],
                    out_hbm.at[c].at[pl.ds(s * RPS, RPS)])


def _tc_dis_u_body(degp_ref, x_ref, dis_ref, u_ref):
    deg = degp_ref[0] + degp_ref[1]
    dis = jnp.where(deg > 0, lax.rsqrt(jnp.maximum(deg, 1e-12)), 0.0)
    dis_ref[...] = dis
    u_ref[...] = x_ref[...] * dis


def _tc_dis_u(degp3, x):
    return pl.pallas_call(
        _tc_dis_u_body,
        grid=(NRB,),
        in_specs=[
            pl.BlockSpec((NC, RB, 1), lambda b: (0, b, 0)),
            pl.BlockSpec((RB, D), lambda b: (b, 0)),
        ],
        out_specs=[
            pl.BlockSpec((RB, 1), lambda b: (b, 0)),
            pl.BlockSpec((RB, D), lambda b: (b, 0)),
        ],
        out_shape=[
            jax.ShapeDtypeStruct((N, 1), jnp.float32),
            jax.ShapeDtypeStruct((N, D), jnp.float32),
        ],
    )(degp3, x)


def _tc_mid_body(pad, gp_ref, dis_ref, tx1_ref, u2_ref):
    g = gp_ref[0] + gp_ref[1]
    dis = dis_ref[...]
    t = -dis * g
    tx1_ref[...] = t
    u2 = dis * t
    if pad:
        u2_ref[...] = jnp.concatenate([u2, jnp.zeros_like(u2)], axis=1)
    else:
        u2_ref[...] = u2


def _tc_mid(gp, dis, F, pad=False):
    FO = 2 * F if pad else F
    return pl.pallas_call(
        functools.partial(_tc_mid_body, pad),
        grid=(NRB,),
        in_specs=[
            pl.BlockSpec((NC, RB, F), lambda b: (0, b, 0)),
            pl.BlockSpec((RB, 1), lambda b: (b, 0)),
        ],
        out_specs=[
            pl.BlockSpec((RB, F), lambda b: (b, 0)),
            pl.BlockSpec((RB, FO), lambda b: (b, 0)),
        ],
        out_shape=[
            jax.ShapeDtypeStruct((N, F), jnp.float32),
            jax.ShapeDtypeStruct((N, FO), jnp.float32),
        ],
    )(gp, dis)


def _tc_fin1_body(z_ref, tx1_ref, g2p_ref, dis_ref, w_ref, b_ref,
                  h_ref, u_ref):
    z = z_ref[...]
    dis = dis_ref[...]
    tx2 = -2.0 * dis * (g2p_ref[0] + g2p_ref[1]) - z
    o = (jnp.dot(z, w_ref[0], preferred_element_type=jnp.float32)
         + jnp.dot(tx1_ref[...], w_ref[1], preferred_element_type=jnp.float32)
         + jnp.dot(tx2, w_ref[2], preferred_element_type=jnp.float32)
         + b_ref[...])
    hh = jnp.maximum(o, 0.0)
    zpad = jnp.zeros_like(hh)
    h_ref[...] = jnp.concatenate([hh, zpad], axis=1)
    u_ref[...] = jnp.concatenate([dis * hh, zpad], axis=1)


def _tc_fin1(x, tx1, g2p, dis, W1, b1):
    return pl.pallas_call(
        _tc_fin1_body,
        grid=(NRB,),
        in_specs=[
            pl.BlockSpec((RB, D), lambda b: (b, 0)),
            pl.BlockSpec((RB, D), lambda b: (b, 0)),
            pl.BlockSpec((NC, RB, D), lambda b: (0, b, 0)),
            pl.BlockSpec((RB, 1), lambda b: (b, 0)),
            pl.BlockSpec((3, D, H), lambda b: (0, 0, 0)),
            pl.BlockSpec((1, H), lambda b: (0, 0)),
        ],
        out_specs=[
            pl.BlockSpec((RB, D), lambda b: (b, 0)),
            pl.BlockSpec((RB, D), lambda b: (b, 0)),
        ],
        out_shape=[
            jax.ShapeDtypeStruct((N, D), jnp.float32),
            jax.ShapeDtypeStruct((N, D), jnp.float32),
        ],
    )(x, tx1, g2p, dis, W1, b1)


def _tc_fin2_body(h_ref, ty1_ref, g2p_ref, dis_ref, w_ref, b_ref, x_ref,
                  xhat_ref, loss_ref):
    h = h_ref[...]
    dis = dis_ref[...]
    ty2 = -2.0 * dis * (g2p_ref[0] + g2p_ref[1]) - h
    o = (jnp.dot(h, w_ref[0], preferred_element_type=jnp.float32)
         + jnp.dot(ty1_ref[...], w_ref[1], preferred_element_type=jnp.float32)
         + jnp.dot(ty2, w_ref[2], preferred_element_type=jnp.float32)
         + b_ref[...])
    xhat_ref[...] = o
    d = o - x_ref[...]
    b_idx = pl.program_id(0)

    @pl.when(b_idx == 0)
    def _():
        loss_ref[...] = jnp.zeros((1, 1), jnp.float32)

    loss_ref[...] += jnp.sum(d * d).reshape(1, 1)

    @pl.when(b_idx == NRB - 1)
    def _():
        loss_ref[...] = loss_ref[...] * (1.0 / (N * D))


def _tc_fin2(h, ty1, g2p, dis, W2, b2, x):
    return pl.pallas_call(
        _tc_fin2_body,
        grid=(NRB,),
        in_specs=[
            pl.BlockSpec((RB, D), lambda b: (b, 0)),
            pl.BlockSpec((RB, D), lambda b: (b, 0)),
            pl.BlockSpec((NC, RB, D), lambda b: (0, b, 0)),
            pl.BlockSpec((RB, 1), lambda b: (b, 0)),
            pl.BlockSpec((3, D, D), lambda b: (0, 0, 0)),
            pl.BlockSpec((1, D), lambda b: (0, 0)),
            pl.BlockSpec((RB, D), lambda b: (b, 0)),
        ],
        out_specs=[
            pl.BlockSpec((RB, D), lambda b: (b, 0)),
            pl.BlockSpec((1, 1), lambda b: (0, 0)),
        ],
        out_shape=[
            jax.ShapeDtypeStruct((N, D), jnp.float32),
            jax.ShapeDtypeStruct((1, 1), jnp.float32),
        ],
    )(h, ty1, g2p, dis, W2, b2, x)


def kernel(x, edge_index, W1, b1, W2, b2):
    src2 = edge_index[0].reshape(NW * NCH, CH)
    dst2 = edge_index[1].reshape(NW * NCH, CH)

    degp = _deg(dst2)
    degp3 = degp[:, :N].reshape(NC, N, 1)
    dis, u = _tc_dis_u(degp3, x)

    g1p = _seg128(u, src2, dst2)
    tx1, u2 = _tc_mid(g1p, dis, D)
    g2p = _seg128(u2, src2, dst2)
    h, uh = _tc_fin1(x, tx1, g2p, dis, W1, b1.reshape(1, H))

    W2p = jnp.concatenate([W2, jnp.zeros((3, D - H, D), jnp.float32)], axis=1)

    q1p = _seg128(uh, src2, dst2)
    ty1, v2 = _tc_mid(q1p, dis, D)
    q2p = _seg128(v2, src2, dst2)
    xhat, lossb = _tc_fin2(h, ty1, q2p, dis, W2p, b2.reshape(1, D), x)

    return (xhat, lossb[0, 0])
